# baseline (speedup 1.0000x reference)
import functools

import jax
import jax.numpy as jnp
from jax import lax
from jax.experimental import pallas as pl
from jax.experimental.pallas import tpu as pltpu
from jax.experimental.pallas import tpu_sc as plsc

_NC = 2
_NS = 16
_NW = _NC * _NS

_BATCH = 16384
_DIM = 32
_PACK = 8
_PROWS = 1000000 // _PACK
_PWORDS = 128
_EW = _DIM // 2
_B_PER_W = _BATCH // _NW
_NSTREAM = 8
_SLEN = _B_PER_W // _NSTREAM


def _gather_body(uj_hbm, ij_hbm, up_hbm, ip_hbm, ue_hbm, ie_hbm,
                 uj_v, ij_v, rows_v, sem):
    wid = lax.axis_index("s") * _NC + lax.axis_index("c")
    base = wid * _B_PER_W
    pltpu.sync_copy(uj_hbm.at[pl.ds(base, _B_PER_W)], uj_v)
    pltpu.sync_copy(ij_hbm.at[pl.ds(base, _B_PER_W)], ij_v)
    for jv, ph, oh in ((uj_v, up_hbm, ue_hbm), (ij_v, ip_hbm, ie_hbm)):
        copies = []
        for k in range(_NSTREAM):
            s = pl.ds(k * _SLEN, _SLEN)
            copies.append(pltpu.async_copy(
                ph.at[jv.at[s]], rows_v.at[s], sem))
        for c in copies:
            c.wait()
        pltpu.sync_copy(rows_v, oh.at[pl.ds(base, _B_PER_W)])


_gather = pl.kernel(
    _gather_body,
    out_type=(
        jax.ShapeDtypeStruct((_BATCH, _PWORDS), jnp.int32),
        jax.ShapeDtypeStruct((_BATCH, _PWORDS), jnp.int32),
    ),
    mesh=plsc.VectorSubcoreMesh(
        core_axis_name="c", subcore_axis_name="s",
        num_cores=_NC, num_subcores=_NS),
    scratch_types=(
        pltpu.VMEM((_B_PER_W,), jnp.int32),
        pltpu.VMEM((_B_PER_W,), jnp.int32),
        pltpu.VMEM((_B_PER_W, _PWORDS), jnp.int32),
        pltpu.SemaphoreType.DMA,
    ),
    compiler_params=pltpu.CompilerParams(use_tc_tiling_on_sc=False),
)

_BB = 1024


def _halves(x):
    hmask = jnp.full(x.shape, -65536, jnp.int32)
    even = lax.bitcast_convert_type(lax.shift_left(x, 16), jnp.float32)
    odd = lax.bitcast_convert_type(lax.bitwise_and(x, hmask), jnp.float32)
    return even, odd


def _mlp_body(ueW_ref, ieW_ref, us_ref, is_ref, w1ue_ref, w1uo_ref,
              w1ie_ref, w1io_ref, b1_ref, w2_ref, b2_ref,
              w3t_ref, b3_ref, out_ref):
    slot = lax.broadcasted_iota(jnp.int32, (1, _PWORDS), 1) // _EW
    mu = (slot == us_ref[...]).astype(jnp.float32)
    mi = (slot == is_ref[...]).astype(jnp.float32)
    ue_e, ue_o = _halves(ueW_ref[...])
    ie_e, ie_o = _halves(ieW_ref[...])
    h = jnp.dot(ue_e * mu, w1ue_ref[...], preferred_element_type=jnp.float32)
    h = h + jnp.dot(ue_o * mu, w1uo_ref[...],
                    preferred_element_type=jnp.float32)
    h = h + jnp.dot(ie_e * mi, w1ie_ref[...],
                    preferred_element_type=jnp.float32)
    h = h + jnp.dot(ie_o * mi, w1io_ref[...],
                    preferred_element_type=jnp.float32)
    h = jnp.maximum(h + b1_ref[...], 0.0)
    h = jnp.maximum(
        jnp.dot(h, w2_ref[...], preferred_element_type=jnp.float32)
        + b2_ref[...], 0.0)
    out_ref[...] = jnp.sum(h * w3t_ref[...], axis=1) + b3_ref[0, 0]


def _mlp(ueW, ieW, us, isx, w1s, b1, w2, b2, w3t, b3):
    grid = _BATCH // _BB
    full = lambda s: pl.BlockSpec(s, lambda i: (0,) * len(s))
    return pl.pallas_call(
        _mlp_body,
        grid=(grid,),
        in_specs=[
            pl.BlockSpec((_BB, _PWORDS), lambda i: (i, 0)),
            pl.BlockSpec((_BB, _PWORDS), lambda i: (i, 0)),
            pl.BlockSpec((_BB, 1), lambda i: (i, 0)),
            pl.BlockSpec((_BB, 1), lambda i: (i, 0)),
            full((_PWORDS, 128)),
            full((_PWORDS, 128)),
            full((_PWORDS, 128)),
            full((_PWORDS, 128)),
            full((1, 128)),
            full((128, 64)),
            full((1, 64)),
            full((1, 64)),
            full((1, 1)),
        ],
        out_specs=pl.BlockSpec((_BB,), lambda i: (i,)),
        out_shape=jax.ShapeDtypeStruct((_BATCH,), jnp.float32),
        compiler_params=pltpu.CompilerParams(
            dimension_semantics=("arbitrary",)),
    )(ueW, ieW, us, isx, *w1s, b1, w2, b2, w3t, b3)


def _pack_table(tab):
    return lax.bitcast_convert_type(
        tab.astype(jnp.bfloat16).reshape(_PROWS, _PWORDS, 2), jnp.int32)


def _w1_rep(w1half):
    even = jnp.tile(w1half[0::2], (_PACK, 1))
    odd = jnp.tile(w1half[1::2], (_PACK, 1))
    return even, odd


@jax.jit
def kernel(user_idx, item_idx, user_table, item_table, W1, b1, W2, b2, W3, b3):
    ui = user_idx.astype(jnp.int32)
    ii = item_idx.astype(jnp.int32)
    up = _pack_table(user_table)
    ip = _pack_table(item_table)
    ueW, ieW = _gather(ui // _PACK, ii // _PACK, up, ip)
    w1ue, w1uo = _w1_rep(W1[:_DIM])
    w1ie, w1io = _w1_rep(W1[_DIM:])
    return _mlp(ueW, ieW, (ui % _PACK).reshape(_BATCH, 1),
                (ii % _PACK).reshape(_BATCH, 1),
                (w1ue, w1uo, w1ie, w1io),
                b1.reshape(1, 128), W2, b2.reshape(1, 64),
                W3.reshape(1, 64), b3.reshape(1, 1))


# speedup vs baseline: 16.6457x; 16.6457x over previous
import functools

import jax
import jax.numpy as jnp
from jax import lax
from jax.experimental import pallas as pl
from jax.experimental.pallas import tpu as pltpu
from jax.experimental.pallas import tpu_sc as plsc

_NC = 2
_NS = 16
_NW = _NC * _NS

_BATCH = 16384
_DIM = 32
_B_PER_W = _BATCH // _NW


def _gather_body(uidx_hbm, iidx_hbm, utab_hbm, itab_hbm, ue_hbm, ie_hbm,
                 uidx_v, urows_v, iidx_v, irows_v, sem_u, sem_i):
    wid = lax.axis_index("s") * _NC + lax.axis_index("c")
    base = wid * _B_PER_W
    pltpu.sync_copy(uidx_hbm.at[pl.ds(base, _B_PER_W)], uidx_v)
    pltpu.sync_copy(iidx_hbm.at[pl.ds(base, _B_PER_W)], iidx_v)
    cu = pltpu.async_copy(utab_hbm.at[uidx_v], urows_v, sem_u)
    ci = pltpu.async_copy(itab_hbm.at[iidx_v], irows_v, sem_i)
    cu.wait()
    ci.wait()
    pltpu.sync_copy(urows_v, ue_hbm.at[pl.ds(base, _B_PER_W)])
    pltpu.sync_copy(irows_v, ie_hbm.at[pl.ds(base, _B_PER_W)])


_gather = pl.kernel(
    _gather_body,
    out_type=(
        jax.ShapeDtypeStruct((_BATCH, _DIM), jnp.float32),
        jax.ShapeDtypeStruct((_BATCH, _DIM), jnp.float32),
    ),
    mesh=plsc.VectorSubcoreMesh(
        core_axis_name="c", subcore_axis_name="s",
        num_cores=_NC, num_subcores=_NS),
    scratch_types=(
        pltpu.VMEM((_B_PER_W,), jnp.int32),
        pltpu.VMEM((_B_PER_W, _DIM), jnp.float32),
        pltpu.VMEM((_B_PER_W,), jnp.int32),
        pltpu.VMEM((_B_PER_W, _DIM), jnp.float32),
        pltpu.SemaphoreType.DMA,
        pltpu.SemaphoreType.DMA,
    ),
    compiler_params=pltpu.CompilerParams(use_tc_tiling_on_sc=False),
)

_BB = 1024


def _mlp_body(ue_ref, ie_ref, w1u_ref, w1i_ref, b1_ref, w2_ref, b2_ref,
              w3t_ref, b3_ref, out_ref):
    h = jnp.dot(ue_ref[...], w1u_ref[...], preferred_element_type=jnp.float32)
    h = h + jnp.dot(ie_ref[...], w1i_ref[...],
                    preferred_element_type=jnp.float32)
    h = jnp.maximum(h + b1_ref[...], 0.0)
    h = jnp.maximum(
        jnp.dot(h, w2_ref[...], preferred_element_type=jnp.float32)
        + b2_ref[...], 0.0)
    out_ref[...] = jnp.sum(h * w3t_ref[...], axis=1) + b3_ref[0, 0]


def _mlp(ue, ie, w1u, w1i, b1, w2, b2, w3t, b3):
    grid = _BATCH // _BB
    full = lambda s: pl.BlockSpec(s, lambda i: (0,) * len(s))
    return pl.pallas_call(
        _mlp_body,
        grid=(grid,),
        in_specs=[
            pl.BlockSpec((_BB, _DIM), lambda i: (i, 0)),
            pl.BlockSpec((_BB, _DIM), lambda i: (i, 0)),
            full((_DIM, 128)),
            full((_DIM, 128)),
            full((1, 128)),
            full((128, 64)),
            full((1, 64)),
            full((1, 64)),
            full((1, 1)),
        ],
        out_specs=pl.BlockSpec((_BB,), lambda i: (i,)),
        out_shape=jax.ShapeDtypeStruct((_BATCH,), jnp.float32),
        compiler_params=pltpu.CompilerParams(
            dimension_semantics=("arbitrary",)),
    )(ue, ie, w1u, w1i, b1, w2, b2, w3t, b3)


@jax.jit
def kernel(user_idx, item_idx, user_table, item_table, W1, b1, W2, b2, W3, b3):
    ue, ie = _gather(user_idx.astype(jnp.int32), item_idx.astype(jnp.int32),
                     user_table, item_table)
    return _mlp(ue, ie, W1[:_DIM], W1[_DIM:], b1.reshape(1, 128),
                W2, b2.reshape(1, 64), W3.reshape(1, 64), b3.reshape(1, 1))
